# Initial kernel scaffold; baseline (speedup 1.0000x reference)
import jax
import jax.numpy as jnp
from jax.experimental import pallas as pl


def kernel(x, edge_index, gene1, gene2, W1, b1, W2, b2, W3, b3, fc1_W, fc1_b, fc2_W, fc2_b):
    raise NotImplementedError("write your pallas kernel here")


# speedup vs baseline: 2.7610x; 2.7610x over previous
import jax
import jax.numpy as jnp
from jax import lax
from jax.experimental import pallas as pl
from jax.experimental.pallas import tpu as pltpu
from jax.experimental.pallas import tpu_sc as plsc

_N = 10000
_NP = 10112
_RPT = 632
_E = 320000
_CW = 128
_CH = 80
_CH2 = 160
_NC = 2
_NS = 16
_NW = _NC * _NS
_EP = _NW * _CH * _CW
_B = 4096
_D = 128


def _sc_mesh():
    return plsc.VectorSubcoreMesh(
        core_axis_name="c", subcore_axis_name="s", num_cores=_NC, num_subcores=_NS
    )


def _zero_rows(rows_v):
    def zfill(i, _):
        for q in range(8):
            rows_v[i, pl.ds(q * 16, 16)] = jnp.zeros((16,), jnp.float32)
        return 0
    lax.fori_loop(0, _CW, zfill, 0)


def _zero_acc_slice(rows_v, acc, base):
    for z in range(4):
        pltpu.sync_copy(rows_v, acc.at[pl.ds(base + z * 128, 128)])
    pltpu.sync_copy(rows_v.at[pl.ds(0, 120)], acc.at[pl.ds(base + 512, 120)])


def _acc_to_hbm(acc, rows_v, out_hbm, c, base):
    for z in range(5):
        rows = 128 if z < 4 else 120
        pltpu.sync_copy(acc.at[pl.ds(base + z * 128, rows)],
                        rows_v.at[pl.ds(0, rows)])
        pltpu.sync_copy(rows_v.at[pl.ds(0, rows)],
                        out_hbm.at[c, pl.ds(base + z * 128, rows)])


def _deg_body(src16, dst16, deg_hbm, idx_v, rows_v, acc):
    c = lax.axis_index("c")
    s = lax.axis_index("s")
    _zero_rows(rows_v)
    base = s * _RPT
    _zero_acc_slice(rows_v, acc, base)

    def ofill(i, _):
        for q in range(8):
            rows_v[i, pl.ds(q * 16, 16)] = jnp.full((16,), 1.0, jnp.float32)
        return 0
    lax.fori_loop(0, _CW, ofill, 0)

    @pl.when(c == 0)
    def _():
        pltpu.sync_copy(src16.at[s], idx_v)

    @pl.when(c == 1)
    def _():
        pltpu.sync_copy(dst16.at[s], idx_v)

    plsc.subcore_barrier()

    def body(j, _):
        pltpu.sync_copy(rows_v, acc.at[idx_v.at[j]], add=True)
        return 0
    lax.fori_loop(0, _CH2, body, 0)
    plsc.subcore_barrier()
    _acc_to_hbm(acc, rows_v, deg_hbm, c, base)


def _deg_call(src16, dst16):
    return pl.kernel(
        _deg_body,
        out_type=jax.ShapeDtypeStruct((_NC, _NP, _D), jnp.float32),
        mesh=_sc_mesh(),
        scratch_types=[
            pltpu.VMEM((_CH2, _CW), jnp.int32),
            pltpu.VMEM((_CW, _D), jnp.float32),
            pltpu.VMEM_SHARED((_NP, _D), jnp.float32),
        ],
    )(src16, dst16)


def _scat_body(h_hbm, src_hbm, dst_hbm, out_hbm, src_v, dst_v, rows_v, acc):
    c = lax.axis_index("c")
    s = lax.axis_index("s")
    wid = c * _NS + s
    pltpu.sync_copy(src_hbm.at[wid], src_v)
    pltpu.sync_copy(dst_hbm.at[wid], dst_v)
    _zero_rows(rows_v)
    base = s * _RPT
    _zero_acc_slice(rows_v, acc, base)
    plsc.subcore_barrier()

    def body(j, _):
        pltpu.sync_copy(h_hbm.at[src_v.at[j]], rows_v)
        pltpu.sync_copy(rows_v, acc.at[dst_v.at[j]], add=True)
        return 0
    lax.fori_loop(0, _CH, body, 0)
    plsc.subcore_barrier()
    _acc_to_hbm(acc, rows_v, out_hbm, c, base)


def _scat_call(h, src3, dst3):
    return pl.kernel(
        _scat_body,
        out_type=jax.ShapeDtypeStruct((_NC, _NP, _D), jnp.float32),
        mesh=_sc_mesh(),
        scratch_types=[
            pltpu.VMEM((_CH, _CW), jnp.int32),
            pltpu.VMEM((_CH, _CW), jnp.int32),
            pltpu.VMEM((_CW, _D), jnp.float32),
            pltpu.VMEM_SHARED((_NP, _D), jnp.float32),
        ],
    )(h, src3, dst3)


def _pair_body(h_hbm, g1_hbm, g2_hbm, o1_hbm, o2_hbm, g_v, rows_v):
    c = lax.axis_index("c")
    s = lax.axis_index("s")
    wid = c * _NS + s
    for g_hbm, o_hbm in ((g1_hbm, o1_hbm), (g2_hbm, o2_hbm)):
        pltpu.sync_copy(g_hbm.at[wid], g_v)
        pltpu.sync_copy(h_hbm.at[g_v], rows_v)
        pltpu.sync_copy(rows_v, o_hbm.at[pl.ds(wid * _CW, _CW)])


def _pair_call(h, g1, g2):
    return pl.kernel(
        _pair_body,
        out_type=(jax.ShapeDtypeStruct((_B, _D), jnp.float32),
                  jax.ShapeDtypeStruct((_B, _D), jnp.float32)),
        mesh=_sc_mesh(),
        scratch_types=[
            pltpu.VMEM((_CW,), jnp.int32),
            pltpu.VMEM((_CW, _D), jnp.float32),
        ],
    )(h, g1, g2)


_BLK = 2528


def _l1_body(x_ref, do_ref, w_ref, o_ref):
    ns = lax.rsqrt(jnp.maximum(do_ref[...], 1.0))
    o_ref[...] = jnp.dot(x_ref[...], w_ref[...],
                         preferred_element_type=jnp.float32) * ns


def _l1_call(x_p, deg_o, w):
    return pl.pallas_call(
        _l1_body,
        grid=(4,),
        in_specs=[
            pl.BlockSpec((_BLK, _D), lambda i: (i, 0)),
            pl.BlockSpec((_BLK, _D), lambda i: (i, 0)),
            pl.BlockSpec((_D, _D), lambda i: (0, 0)),
        ],
        out_specs=pl.BlockSpec((_BLK, _D), lambda i: (i, 0)),
        out_shape=jax.ShapeDtypeStruct((_NP, _D), jnp.float32),
    )(x_p, deg_o, w)


def _l23_body(s_ref, di_ref, do_ref, b_ref, w_ref, o_ref):
    nd = lax.rsqrt(jnp.maximum(di_ref[...], 1.0))
    ns = lax.rsqrt(jnp.maximum(do_ref[...], 1.0))
    h = jnp.maximum((s_ref[0] + s_ref[1]) * nd + b_ref[...], 0.0)
    o_ref[...] = jnp.dot(h, w_ref[...],
                         preferred_element_type=jnp.float32) * ns


def _l23_call(s, deg_i, deg_o, b, w):
    return pl.pallas_call(
        _l23_body,
        grid=(4,),
        in_specs=[
            pl.BlockSpec((_NC, _BLK, _D), lambda i: (0, i, 0)),
            pl.BlockSpec((_BLK, _D), lambda i: (i, 0)),
            pl.BlockSpec((_BLK, _D), lambda i: (i, 0)),
            pl.BlockSpec((1, _D), lambda i: (0, 0)),
            pl.BlockSpec((_D, _D), lambda i: (0, 0)),
        ],
        out_specs=pl.BlockSpec((_BLK, _D), lambda i: (i, 0)),
        out_shape=jax.ShapeDtypeStruct((_NP, _D), jnp.float32),
    )(s, deg_i, deg_o, b, w)


def _fin_body(s_ref, di_ref, b_ref, o_ref):
    nd = lax.rsqrt(jnp.maximum(di_ref[...], 1.0))
    o_ref[...] = (s_ref[0] + s_ref[1]) * nd + b_ref[...]


def _fin_call(s, deg_i, b):
    return pl.pallas_call(
        _fin_body,
        grid=(4,),
        in_specs=[
            pl.BlockSpec((_NC, _BLK, _D), lambda i: (0, i, 0)),
            pl.BlockSpec((_BLK, _D), lambda i: (i, 0)),
            pl.BlockSpec((1, _D), lambda i: (0, 0)),
        ],
        out_specs=pl.BlockSpec((_BLK, _D), lambda i: (i, 0)),
        out_shape=jax.ShapeDtypeStruct((_NP, _D), jnp.float32),
    )(s, deg_i, b)


def _mlp_body(g1_ref, g2_ref, a_ref, bm_ref, b1_ref, w2_ref, b2_ref, o_ref):
    z = jnp.dot(g1_ref[...], a_ref[...], preferred_element_type=jnp.float32)
    z = z + jnp.dot(g2_ref[...], bm_ref[...], preferred_element_type=jnp.float32)
    z = jnp.maximum(z + b1_ref[...], 0.0)
    lp = jnp.dot(z, w2_ref[...], preferred_element_type=jnp.float32) + b2_ref[...]
    l0 = lp[:, 0:1]
    l1 = lp[:, 1:2]
    m = jnp.maximum(l0, l1)
    e0 = jnp.exp(l0 - m)
    e1 = jnp.exp(l1 - m)
    tot = e0 + e1
    col = lax.broadcasted_iota(jnp.int32, (_B, _D), 1)
    o_ref[...] = jnp.where(col == 0, e0 / tot, jnp.where(col == 1, e1 / tot, 0.0))


def _mlp_call(r1, r2, a, bm, b1, w2p, b2p):
    return pl.pallas_call(
        _mlp_body,
        out_shape=jax.ShapeDtypeStruct((_B, _D), jnp.float32),
    )(r1, r2, a, bm, b1, w2p, b2p)


def kernel(x, edge_index, gene1, gene2, W1, b1, W2, b2, W3, b3,
           fc1_W, fc1_b, fc2_W, fc2_b):
    src = edge_index[0]
    dst = edge_index[1]
    src_p = jnp.pad(src, (0, _EP - _E), constant_values=_N)
    dst_p = jnp.pad(dst, (0, _EP - _E), constant_values=_N)
    src3 = src_p.reshape(_NW, _CH, _CW)
    dst3 = dst_p.reshape(_NW, _CH, _CW)
    src16 = src_p.reshape(_NS, _CH2, _CW)
    dst16 = dst_p.reshape(_NS, _CH2, _CW)
    x_p = jnp.pad(x, ((0, _NP - _N), (0, 0)))
    g1 = gene1.reshape(_NW, _CW)
    g2 = gene2.reshape(_NW, _CW)

    deg = _deg_call(src16, dst16)
    deg_o = deg[0]
    deg_i = deg[1]

    h = _l1_call(x_p, deg_o, W1)
    s1 = _scat_call(h, src3, dst3)
    h = _l23_call(s1, deg_i, deg_o, b1.reshape(1, _D), W2)
    s2 = _scat_call(h, src3, dst3)
    h = _l23_call(s2, deg_i, deg_o, b2.reshape(1, _D), W3)
    s3 = _scat_call(h, src3, dst3)
    hf = _fin_call(s3, deg_i, b3.reshape(1, _D))

    r1, r2 = _pair_call(hf, g1, g2)

    a = fc1_W[:_D]
    bm = fc1_W[_D:]
    w2p = jnp.zeros((_D, _D), jnp.float32).at[:, :2].set(fc2_W)
    b2p = jnp.zeros((_D,), jnp.float32).at[:2].set(fc2_b)
    probs_p = _mlp_call(r1, r2, a, bm, fc1_b.reshape(1, _D),
                        w2p, b2p.reshape(1, _D))
    return probs_p[:, :2]


# speedup vs baseline: 3.0553x; 1.1066x over previous
import jax
import jax.numpy as jnp
from jax import lax
from jax.experimental import pallas as pl
from jax.experimental.pallas import tpu as pltpu
from jax.experimental.pallas import tpu_sc as plsc

_N = 10000
_NP = 10112
_RPT = 632
_E = 320000
_CW = 128
_CH2 = 160
_NC = 2
_NS = 16
_NW = _NC * _NS
_EP = 327680
_B = 4096
_D = 128


def _sc_mesh():
    return plsc.VectorSubcoreMesh(
        core_axis_name="c", subcore_axis_name="s", num_cores=_NC, num_subcores=_NS
    )


def _zero_rows(rows_v, n):
    def zfill(i, _):
        for q in range(8):
            rows_v[i, pl.ds(q * 16, 16)] = jnp.zeros((16,), jnp.float32)
        return 0
    lax.fori_loop(0, n, zfill, 0)


def _chunks(r):
    full, tail = divmod(_RPT, r)
    sizes = [r] * full + ([tail] if tail else [])
    offs, o = [], 0
    for sz in sizes:
        offs.append(o)
        o += sz
    return list(zip(offs, sizes))


def _zero_acc_slice(rows_v, r, acc, base):
    for off, sz in _chunks(r):
        pltpu.sync_copy(rows_v.at[pl.ds(0, sz)], acc.at[pl.ds(base + off, sz)])


def _acc_to_hbm(acc, rows_v, r, out_hbm, c, base):
    for off, sz in _chunks(r):
        pltpu.sync_copy(acc.at[pl.ds(base + off, sz)], rows_v.at[pl.ds(0, sz)])
        pltpu.sync_copy(rows_v.at[pl.ds(0, sz)],
                        out_hbm.at[c, pl.ds(base + off, sz)])


def _deg_body(src16, dst16, deg_hbm, idx_v, rows_v, acc, sem):
    c = lax.axis_index("c")
    s = lax.axis_index("s")
    _zero_rows(rows_v, _CW)
    base = s * _RPT
    _zero_acc_slice(rows_v, _CW, acc, base)

    def ofill(i, _):
        for q in range(8):
            rows_v[i, pl.ds(q * 16, 16)] = jnp.full((16,), 1.0, jnp.float32)
        return 0
    lax.fori_loop(0, _CW, ofill, 0)

    @pl.when(c == 0)
    def _():
        pltpu.sync_copy(src16.at[s], idx_v)

    @pl.when(c == 1)
    def _():
        pltpu.sync_copy(dst16.at[s], idx_v)

    plsc.subcore_barrier()

    def body(j, _):
        pltpu.sync_copy(rows_v, acc.at[idx_v.at[j]], add=True)
        return 0
    lax.fori_loop(0, _CH2, body, 0)
    plsc.subcore_barrier()
    _acc_to_hbm(acc, rows_v, _CW, deg_hbm, c, base)


def _deg_call(src16, dst16):
    return pl.kernel(
        _deg_body,
        out_type=jax.ShapeDtypeStruct((_NC, _NP, _D), jnp.float32),
        mesh=_sc_mesh(),
        scratch_types=[
            pltpu.VMEM((_CH2, _CW), jnp.int32),
            pltpu.VMEM((_CW, _D), jnp.float32),
            pltpu.VMEM_SHARED((_NP, _D), jnp.float32),
            pltpu.SemaphoreType.DMA,
        ],
    )(src16, dst16)


_GW = 64
_GN = 160
_PAIRS = 80
_EPT = _GN * _GW


def _scat_body(h_hbm, src_hbm, dst_hbm, out_hbm, rows_big, ib, db, acc,
               sem_i, sem_g, sem_d):
    c = lax.axis_index("c")
    s = lax.axis_index("s")
    wid = c * _NS + s
    ebase = wid * _EPT
    _zero_rows(rows_big, 128)
    base = s * _RPT
    _zero_acc_slice(rows_big, 128, acc, base)

    pltpu.async_copy(dst_hbm.at[wid].at[0], db.at[0], sem_d)
    pltpu.async_copy(dst_hbm.at[wid].at[1], db.at[1], sem_d)
    plsc.subcore_barrier()

    def step(t, _):
        @pl.when(t < _GN)
        def _():
            slot = lax.rem(t, 6)
            pltpu.async_copy(src_hbm.at[pl.ds(ebase + t * _GW, _GW)],
                             ib.at[pl.ds(slot * _GW, _GW)], sem_i)

        @pl.when(jnp.logical_and(t >= 2, t < _GN + 2))
        def _():
            j = t - 2
            islot = lax.rem(j, 6)
            gslot = lax.rem(j, 4)
            pltpu.make_async_copy(src_hbm.at[pl.ds(0, _GW)],
                                  ib.at[pl.ds(0, _GW)], sem_i).wait()
            pltpu.async_copy(h_hbm.at[ib.at[pl.ds(islot * _GW, _GW)]],
                             rows_big.at[pl.ds(gslot * _GW, _GW)], sem_g)

        @pl.when(jnp.logical_and(t >= 5, lax.rem(t, 2) == 1))
        def _():
            m = (t - 5) // 2
            ms = lax.rem(m, 2)
            pltpu.make_async_copy(h_hbm.at[pl.ds(0, 128)],
                                  rows_big.at[pl.ds(0, 128)], sem_g).wait()
            pltpu.make_async_copy(dst_hbm.at[wid].at[0], db.at[0], sem_d).wait()
            pltpu.sync_copy(rows_big.at[pl.ds(ms * 128, 128)],
                            acc.at[db.at[ms]], add=True)

            @pl.when(m + 2 < _PAIRS)
            def _():
                pltpu.async_copy(dst_hbm.at[wid].at[m + 2], db.at[ms], sem_d)
        return 0
    lax.fori_loop(0, 2 * _PAIRS + 4, step, 0)
    plsc.subcore_barrier()
    _acc_to_hbm(acc, rows_big, 128, out_hbm, c, base)


def _scat_call(h, src1, dst3):
    return pl.kernel(
        _scat_body,
        out_type=jax.ShapeDtypeStruct((_NC, _NP, _D), jnp.float32),
        mesh=_sc_mesh(),
        scratch_types=[
            pltpu.VMEM((4 * _GW, _D), jnp.float32),
            pltpu.VMEM((6 * _GW,), jnp.int32),
            pltpu.VMEM((2, 128), jnp.int32),
            pltpu.VMEM_SHARED((_NP, _D), jnp.float32),
            pltpu.SemaphoreType.DMA,
            pltpu.SemaphoreType.DMA,
            pltpu.SemaphoreType.DMA,
        ],
    )(h, src1, dst3)


def _pair_body(h_hbm, g1_hbm, g2_hbm, o1_hbm, o2_hbm, g_v, rows_v):
    c = lax.axis_index("c")
    s = lax.axis_index("s")
    wid = c * _NS + s
    for g_hbm, o_hbm in ((g1_hbm, o1_hbm), (g2_hbm, o2_hbm)):
        pltpu.sync_copy(g_hbm.at[wid], g_v)
        pltpu.sync_copy(h_hbm.at[g_v], rows_v)
        pltpu.sync_copy(rows_v, o_hbm.at[pl.ds(wid * _CW, _CW)])


def _pair_call(h, g1, g2):
    return pl.kernel(
        _pair_body,
        out_type=(jax.ShapeDtypeStruct((_B, _D), jnp.float32),
                  jax.ShapeDtypeStruct((_B, _D), jnp.float32)),
        mesh=_sc_mesh(),
        scratch_types=[
            pltpu.VMEM((_CW,), jnp.int32),
            pltpu.VMEM((_CW, _D), jnp.float32),
        ],
    )(h, g1, g2)


_BLK = 2528


def _l1_body(x_ref, do_ref, w_ref, o_ref):
    ns = lax.rsqrt(jnp.maximum(do_ref[...], 1.0))
    o_ref[...] = jnp.dot(x_ref[...], w_ref[...],
                         preferred_element_type=jnp.float32) * ns


def _l1_call(x_p, deg_o, w):
    return pl.pallas_call(
        _l1_body,
        grid=(4,),
        in_specs=[
            pl.BlockSpec((_BLK, _D), lambda i: (i, 0)),
            pl.BlockSpec((_BLK, _D), lambda i: (i, 0)),
            pl.BlockSpec((_D, _D), lambda i: (0, 0)),
        ],
        out_specs=pl.BlockSpec((_BLK, _D), lambda i: (i, 0)),
        out_shape=jax.ShapeDtypeStruct((_NP, _D), jnp.float32),
    )(x_p, deg_o, w)


def _l23_body(s_ref, di_ref, do_ref, b_ref, w_ref, o_ref):
    nd = lax.rsqrt(jnp.maximum(di_ref[...], 1.0))
    ns = lax.rsqrt(jnp.maximum(do_ref[...], 1.0))
    h = jnp.maximum((s_ref[0] + s_ref[1]) * nd + b_ref[...], 0.0)
    o_ref[...] = jnp.dot(h, w_ref[...],
                         preferred_element_type=jnp.float32) * ns


def _l23_call(s, deg_i, deg_o, b, w):
    return pl.pallas_call(
        _l23_body,
        grid=(4,),
        in_specs=[
            pl.BlockSpec((_NC, _BLK, _D), lambda i: (0, i, 0)),
            pl.BlockSpec((_BLK, _D), lambda i: (i, 0)),
            pl.BlockSpec((_BLK, _D), lambda i: (i, 0)),
            pl.BlockSpec((1, _D), lambda i: (0, 0)),
            pl.BlockSpec((_D, _D), lambda i: (0, 0)),
        ],
        out_specs=pl.BlockSpec((_BLK, _D), lambda i: (i, 0)),
        out_shape=jax.ShapeDtypeStruct((_NP, _D), jnp.float32),
    )(s, deg_i, deg_o, b, w)


def _fin_body(s_ref, di_ref, b_ref, o_ref):
    nd = lax.rsqrt(jnp.maximum(di_ref[...], 1.0))
    o_ref[...] = (s_ref[0] + s_ref[1]) * nd + b_ref[...]


def _fin_call(s, deg_i, b):
    return pl.pallas_call(
        _fin_body,
        grid=(4,),
        in_specs=[
            pl.BlockSpec((_NC, _BLK, _D), lambda i: (0, i, 0)),
            pl.BlockSpec((_BLK, _D), lambda i: (i, 0)),
            pl.BlockSpec((1, _D), lambda i: (0, 0)),
        ],
        out_specs=pl.BlockSpec((_BLK, _D), lambda i: (i, 0)),
        out_shape=jax.ShapeDtypeStruct((_NP, _D), jnp.float32),
    )(s, deg_i, b)


def _mlp_body(g1_ref, g2_ref, a_ref, bm_ref, b1_ref, w2_ref, b2_ref, o_ref):
    z = jnp.dot(g1_ref[...], a_ref[...], preferred_element_type=jnp.float32)
    z = z + jnp.dot(g2_ref[...], bm_ref[...], preferred_element_type=jnp.float32)
    z = jnp.maximum(z + b1_ref[...], 0.0)
    lp = jnp.dot(z, w2_ref[...], preferred_element_type=jnp.float32) + b2_ref[...]
    l0 = lp[:, 0:1]
    l1 = lp[:, 1:2]
    m = jnp.maximum(l0, l1)
    e0 = jnp.exp(l0 - m)
    e1 = jnp.exp(l1 - m)
    tot = e0 + e1
    col = lax.broadcasted_iota(jnp.int32, (_B, _D), 1)
    o_ref[...] = jnp.where(col == 0, e0 / tot, jnp.where(col == 1, e1 / tot, 0.0))


def _mlp_call(r1, r2, a, bm, b1, w2p, b2p):
    return pl.pallas_call(
        _mlp_body,
        out_shape=jax.ShapeDtypeStruct((_B, _D), jnp.float32),
    )(r1, r2, a, bm, b1, w2p, b2p)


def kernel(x, edge_index, gene1, gene2, W1, b1, W2, b2, W3, b3,
           fc1_W, fc1_b, fc2_W, fc2_b):
    src = edge_index[0]
    dst = edge_index[1]
    src_p = jnp.pad(src, (0, _EP - _E), constant_values=_N)
    dst_p = jnp.pad(dst, (0, _EP - _E), constant_values=_N)
    src1 = src_p
    dst3 = dst_p.reshape(_NW, _PAIRS, _CW)
    src16 = src_p.reshape(_NS, _CH2, _CW)
    dst16 = dst_p.reshape(_NS, _CH2, _CW)
    x_p = jnp.pad(x, ((0, _NP - _N), (0, 0)))
    g1 = gene1.reshape(_NW, _CW)
    g2 = gene2.reshape(_NW, _CW)

    deg = _deg_call(src16, dst16)
    deg_o = deg[0]
    deg_i = deg[1]

    h = _l1_call(x_p, deg_o, W1)
    s1 = _scat_call(h, src1, dst3)
    h = _l23_call(s1, deg_i, deg_o, b1.reshape(1, _D), W2)
    s2 = _scat_call(h, src1, dst3)
    h = _l23_call(s2, deg_i, deg_o, b2.reshape(1, _D), W3)
    s3 = _scat_call(h, src1, dst3)
    hf = _fin_call(s3, deg_i, b3.reshape(1, _D))

    r1, r2 = _pair_call(hf, g1, g2)

    a = fc1_W[:_D]
    bm = fc1_W[_D:]
    w2p = jnp.zeros((_D, _D), jnp.float32).at[:, :2].set(fc2_W)
    b2p = jnp.zeros((_D,), jnp.float32).at[:2].set(fc2_b)
    probs_p = _mlp_call(r1, r2, a, bm, fc1_b.reshape(1, _D),
                        w2p, b2p.reshape(1, _D))
    return probs_p[:, :2]


# speedup vs baseline: 10.5839x; 3.4641x over previous
import jax
import jax.numpy as jnp
from jax import lax
from jax.experimental import pallas as pl
from jax.experimental.pallas import tpu as pltpu
from jax.experimental.pallas import tpu_sc as plsc

_N = 10000
_NP = 10112
_RPT = 632
_E = 320000
_CW = 128
_CH2 = 160
_NC = 2
_NS = 16
_NW = _NC * _NS
_EP = 327680
_B = 4096
_D = 128


def _sc_mesh():
    return plsc.VectorSubcoreMesh(
        core_axis_name="c", subcore_axis_name="s", num_cores=_NC, num_subcores=_NS
    )


def _zero_rows(rows_v, n):
    def zfill(i, _):
        for q in range(8):
            rows_v[i, pl.ds(q * 16, 16)] = jnp.zeros((16,), jnp.float32)
        return 0
    lax.fori_loop(0, n, zfill, 0)


def _chunks(r):
    full, tail = divmod(_RPT, r)
    sizes = [r] * full + ([tail] if tail else [])
    offs, o = [], 0
    for sz in sizes:
        offs.append(o)
        o += sz
    return list(zip(offs, sizes))


def _zero_acc_slice(rows_v, r, acc, base):
    for off, sz in _chunks(r):
        pltpu.sync_copy(rows_v.at[pl.ds(0, sz)], acc.at[pl.ds(base + off, sz)])


def _acc_to_hbm(acc, rows_v, r, out_hbm, c, base):
    for off, sz in _chunks(r):
        pltpu.sync_copy(acc.at[pl.ds(base + off, sz)], rows_v.at[pl.ds(0, sz)])
        pltpu.sync_copy(rows_v.at[pl.ds(0, sz)],
                        out_hbm.at[c, pl.ds(base + off, sz)])


def _deg_body(src16, dst16, deg_hbm, idx_v, rows_v, acc, sem):
    c = lax.axis_index("c")
    s = lax.axis_index("s")
    _zero_rows(rows_v, _CW)
    base = s * _RPT
    _zero_acc_slice(rows_v, _CW, acc, base)

    def ofill(i, _):
        for q in range(8):
            rows_v[i, pl.ds(q * 16, 16)] = jnp.full((16,), 1.0, jnp.float32)
        return 0
    lax.fori_loop(0, _CW, ofill, 0)

    @pl.when(c == 0)
    def _():
        pltpu.sync_copy(src16.at[s], idx_v)

    @pl.when(c == 1)
    def _():
        pltpu.sync_copy(dst16.at[s], idx_v)

    plsc.subcore_barrier()

    def body(j, _):
        pltpu.sync_copy(rows_v, acc.at[idx_v.at[j]], add=True)
        return 0
    lax.fori_loop(0, _CH2, body, 0)
    plsc.subcore_barrier()
    _acc_to_hbm(acc, rows_v, _CW, deg_hbm, c, base)


def _deg_call(src16, dst16):
    return pl.kernel(
        _deg_body,
        out_type=jax.ShapeDtypeStruct((_NC, _NP, _D), jnp.float32),
        mesh=_sc_mesh(),
        scratch_types=[
            pltpu.VMEM((_CH2, _CW), jnp.int32),
            pltpu.VMEM((_CW, _D), jnp.float32),
            pltpu.VMEM_SHARED((_NP, _D), jnp.float32),
            pltpu.SemaphoreType.DMA,
        ],
    )(src16, dst16)


_GW = 64
_GN = 160
_PAIRS = 80
_EPT = _GN * _GW


def _scat_body(h_hbm, src_hbm, dst_hbm, out_hbm, rows_big, ib, db, acc,
               sem_i, sem_g, sem_d):
    c = lax.axis_index("c")
    s = lax.axis_index("s")
    wid = c * _NS + s
    ebase = wid * _EPT
    _zero_rows(rows_big, 128)
    base = s * _RPT
    _zero_acc_slice(rows_big, 128, acc, base)

    pltpu.async_copy(dst_hbm.at[wid].at[0], db.at[0], sem_d)
    pltpu.async_copy(dst_hbm.at[wid].at[1], db.at[1], sem_d)
    plsc.subcore_barrier()

    def step(t, _):
        @pl.when(t < _GN)
        def _():
            slot = lax.rem(t, 6)
            pltpu.async_copy(src_hbm.at[pl.ds(ebase + t * _GW, _GW)],
                             ib.at[pl.ds(slot * _GW, _GW)], sem_i)

        @pl.when(jnp.logical_and(t >= 2, t < _GN + 2))
        def _():
            j = t - 2
            islot = lax.rem(j, 6)
            gslot = lax.rem(j, 4)
            pltpu.make_async_copy(src_hbm.at[pl.ds(0, _GW)],
                                  ib.at[pl.ds(0, _GW)], sem_i).wait()
            pltpu.async_copy(h_hbm.at[ib.at[pl.ds(islot * _GW, _GW)]],
                             rows_big.at[pl.ds(gslot * _GW, _GW)], sem_g)

        @pl.when(jnp.logical_and(t >= 5, lax.rem(t, 2) == 1))
        def _():
            m = (t - 5) // 2
            ms = lax.rem(m, 2)
            pltpu.make_async_copy(h_hbm.at[pl.ds(0, 128)],
                                  rows_big.at[pl.ds(0, 128)], sem_g).wait()
            pltpu.make_async_copy(dst_hbm.at[wid].at[0], db.at[0], sem_d).wait()
            pltpu.sync_copy(rows_big.at[pl.ds(ms * 128, 128)],
                            acc.at[db.at[ms]], add=True)

            @pl.when(m + 2 < _PAIRS)
            def _():
                pltpu.async_copy(dst_hbm.at[wid].at[m + 2], db.at[ms], sem_d)
        return 0
    lax.fori_loop(0, 2 * _PAIRS + 4, step, 0)
    plsc.subcore_barrier()
    _acc_to_hbm(acc, rows_big, 128, out_hbm, c, base)


def _scat_call(h, src1, dst3):
    return pl.kernel(
        _scat_body,
        out_type=jax.ShapeDtypeStruct((_NC, _NP, _D), jnp.float32),
        mesh=_sc_mesh(),
        scratch_types=[
            pltpu.VMEM((4 * _GW, _D), jnp.float32),
            pltpu.VMEM((6 * _GW,), jnp.int32),
            pltpu.VMEM((2, 128), jnp.int32),
            pltpu.VMEM_SHARED((_NP, _D), jnp.float32),
            pltpu.SemaphoreType.DMA,
            pltpu.SemaphoreType.DMA,
            pltpu.SemaphoreType.DMA,
        ],
    )(h, src1, dst3)


def _pair_body(h_hbm, g1_hbm, g2_hbm, o1_hbm, o2_hbm, g_v, rows_v):
    c = lax.axis_index("c")
    s = lax.axis_index("s")
    wid = c * _NS + s
    for g_hbm, o_hbm in ((g1_hbm, o1_hbm), (g2_hbm, o2_hbm)):
        pltpu.sync_copy(g_hbm.at[wid], g_v)
        pltpu.sync_copy(h_hbm.at[g_v], rows_v)
        pltpu.sync_copy(rows_v, o_hbm.at[pl.ds(wid * _CW, _CW)])


def _pair_call(h, g1, g2):
    return pl.kernel(
        _pair_body,
        out_type=(jax.ShapeDtypeStruct((_B, _D), jnp.float32),
                  jax.ShapeDtypeStruct((_B, _D), jnp.float32)),
        mesh=_sc_mesh(),
        scratch_types=[
            pltpu.VMEM((_CW,), jnp.int32),
            pltpu.VMEM((_CW, _D), jnp.float32),
        ],
    )(h, g1, g2)


_BLK = 2528


def _l1_body(x_ref, do_ref, w_ref, o_ref):
    ns = lax.rsqrt(jnp.maximum(do_ref[...], 1.0))
    o_ref[...] = jnp.dot(x_ref[...], w_ref[...],
                         preferred_element_type=jnp.float32) * ns


def _l1_call(x_p, deg_o, w):
    return pl.pallas_call(
        _l1_body,
        grid=(4,),
        in_specs=[
            pl.BlockSpec((_BLK, _D), lambda i: (i, 0)),
            pl.BlockSpec((_BLK, _D), lambda i: (i, 0)),
            pl.BlockSpec((_D, _D), lambda i: (0, 0)),
        ],
        out_specs=pl.BlockSpec((_BLK, _D), lambda i: (i, 0)),
        out_shape=jax.ShapeDtypeStruct((_NP, _D), jnp.float32),
    )(x_p, deg_o, w)


def _l23_body(s_ref, di_ref, do_ref, b_ref, w_ref, o_ref):
    nd = lax.rsqrt(jnp.maximum(di_ref[...], 1.0))
    ns = lax.rsqrt(jnp.maximum(do_ref[...], 1.0))
    h = jnp.maximum((s_ref[0] + s_ref[1]) * nd + b_ref[...], 0.0)
    o_ref[...] = jnp.dot(h, w_ref[...],
                         preferred_element_type=jnp.float32) * ns


def _l23_call(s, deg_i, deg_o, b, w):
    return pl.pallas_call(
        _l23_body,
        grid=(4,),
        in_specs=[
            pl.BlockSpec((_NC, _BLK, _D), lambda i: (0, i, 0)),
            pl.BlockSpec((_BLK, _D), lambda i: (i, 0)),
            pl.BlockSpec((_BLK, _D), lambda i: (i, 0)),
            pl.BlockSpec((1, _D), lambda i: (0, 0)),
            pl.BlockSpec((_D, _D), lambda i: (0, 0)),
        ],
        out_specs=pl.BlockSpec((_BLK, _D), lambda i: (i, 0)),
        out_shape=jax.ShapeDtypeStruct((_NP, _D), jnp.float32),
    )(s, deg_i, deg_o, b, w)


def _fin_body(s_ref, di_ref, b_ref, o_ref):
    nd = lax.rsqrt(jnp.maximum(di_ref[...], 1.0))
    o_ref[...] = (s_ref[0] + s_ref[1]) * nd + b_ref[...]


def _fin_call(s, deg_i, b):
    return pl.pallas_call(
        _fin_body,
        grid=(4,),
        in_specs=[
            pl.BlockSpec((_NC, _BLK, _D), lambda i: (0, i, 0)),
            pl.BlockSpec((_BLK, _D), lambda i: (i, 0)),
            pl.BlockSpec((1, _D), lambda i: (0, 0)),
        ],
        out_specs=pl.BlockSpec((_BLK, _D), lambda i: (i, 0)),
        out_shape=jax.ShapeDtypeStruct((_NP, _D), jnp.float32),
    )(s, deg_i, b)


def _mlp_body(g1_ref, g2_ref, a_ref, bm_ref, b1_ref, w2_ref, b2_ref, o_ref):
    z = jnp.dot(g1_ref[...], a_ref[...], preferred_element_type=jnp.float32)
    z = z + jnp.dot(g2_ref[...], bm_ref[...], preferred_element_type=jnp.float32)
    z = jnp.maximum(z + b1_ref[...], 0.0)
    lp = jnp.dot(z, w2_ref[...], preferred_element_type=jnp.float32) + b2_ref[...]
    l0 = lp[:, 0:1]
    l1 = lp[:, 1:2]
    m = jnp.maximum(l0, l1)
    e0 = jnp.exp(l0 - m)
    e1 = jnp.exp(l1 - m)
    tot = e0 + e1
    col = lax.broadcasted_iota(jnp.int32, (_B, _D), 1)
    o_ref[...] = jnp.where(col == 0, e0 / tot, jnp.where(col == 1, e1 / tot, 0.0))


def _mlp_call(r1, r2, a, bm, b1, w2p, b2p):
    return pl.pallas_call(
        _mlp_body,
        out_shape=jax.ShapeDtypeStruct((_B, _D), jnp.float32),
    )(r1, r2, a, bm, b1, w2p, b2p)


def kernel(x, edge_index, gene1, gene2, W1, b1, W2, b2, W3, b3,
           fc1_W, fc1_b, fc2_W, fc2_b):
    src = edge_index[0]
    dst = edge_index[1]
    pad_idx = _N + jnp.arange(_EP - _E, dtype=jnp.int32) % (_NP - _N)
    src_p = jnp.concatenate([src, pad_idx])
    dst_p = jnp.concatenate([dst, pad_idx])
    src1 = src_p
    dst3 = dst_p.reshape(_NW, _PAIRS, _CW)
    src16 = src_p.reshape(_NS, _CH2, _CW)
    dst16 = dst_p.reshape(_NS, _CH2, _CW)
    x_p = jnp.pad(x, ((0, _NP - _N), (0, 0)))
    g1 = gene1.reshape(_NW, _CW)
    g2 = gene2.reshape(_NW, _CW)

    deg = _deg_call(src16, dst16)
    deg_o = deg[0]
    deg_i = deg[1]

    h = _l1_call(x_p, deg_o, W1)
    s1 = _scat_call(h, src1, dst3)
    h = _l23_call(s1, deg_i, deg_o, b1.reshape(1, _D), W2)
    s2 = _scat_call(h, src1, dst3)
    h = _l23_call(s2, deg_i, deg_o, b2.reshape(1, _D), W3)
    s3 = _scat_call(h, src1, dst3)
    hf = _fin_call(s3, deg_i, b3.reshape(1, _D))

    r1, r2 = _pair_call(hf, g1, g2)

    a = fc1_W[:_D]
    bm = fc1_W[_D:]
    w2p = jnp.zeros((_D, _D), jnp.float32).at[:, :2].set(fc2_W)
    b2p = jnp.zeros((_D,), jnp.float32).at[:2].set(fc2_b)
    probs_p = _mlp_call(r1, r2, a, bm, fc1_b.reshape(1, _D),
                        w2p, b2p.reshape(1, _D))
    return probs_p[:, :2]


# speedup vs baseline: 10.6318x; 1.0045x over previous
import jax
import jax.numpy as jnp
from jax import lax
from jax.experimental import pallas as pl
from jax.experimental.pallas import tpu as pltpu
from jax.experimental.pallas import tpu_sc as plsc

_N = 10000
_NP = 10112
_RPT = 632
_E = 320000
_CW = 128
_CH2 = 160
_NC = 2
_NS = 16
_NW = _NC * _NS
_EP = 327680
_B = 4096
_D = 128


def _sc_mesh():
    return plsc.VectorSubcoreMesh(
        core_axis_name="c", subcore_axis_name="s", num_cores=_NC, num_subcores=_NS
    )


def _zero_rows(rows_v, n):
    def zfill(i, _):
        for q in range(8):
            rows_v[i, pl.ds(q * 16, 16)] = jnp.zeros((16,), jnp.float32)
        return 0
    lax.fori_loop(0, n, zfill, 0)


def _chunks(r):
    full, tail = divmod(_RPT, r)
    sizes = [r] * full + ([tail] if tail else [])
    offs, o = [], 0
    for sz in sizes:
        offs.append(o)
        o += sz
    return list(zip(offs, sizes))


def _zero_acc_slice(rows_v, r, acc, base):
    for off, sz in _chunks(r):
        pltpu.sync_copy(rows_v.at[pl.ds(0, sz)], acc.at[pl.ds(base + off, sz)])


def _acc_to_hbm(acc, rows_v, r, out_hbm, c, base):
    for off, sz in _chunks(r):
        pltpu.sync_copy(acc.at[pl.ds(base + off, sz)], rows_v.at[pl.ds(0, sz)])
        pltpu.sync_copy(rows_v.at[pl.ds(0, sz)],
                        out_hbm.at[c, pl.ds(base + off, sz)])


def _deg_body(src16, dst16, deg_hbm, idx_v, rows_v, acc, sem):
    c = lax.axis_index("c")
    s = lax.axis_index("s")
    _zero_rows(rows_v, _CW)
    base = s * _RPT
    _zero_acc_slice(rows_v, _CW, acc, base)

    def ofill(i, _):
        for q in range(8):
            rows_v[i, pl.ds(q * 16, 16)] = jnp.full((16,), 1.0, jnp.float32)
        return 0
    lax.fori_loop(0, _CW, ofill, 0)

    @pl.when(c == 0)
    def _():
        pltpu.sync_copy(src16.at[s], idx_v)

    @pl.when(c == 1)
    def _():
        pltpu.sync_copy(dst16.at[s], idx_v)

    plsc.subcore_barrier()

    def body(j, _):
        pltpu.async_copy(rows_v, acc.at[idx_v.at[j]], sem, add=True)

        @pl.when(j >= 8)
        def _():
            pltpu.make_async_copy(rows_v, acc.at[idx_v.at[0]], sem).wait()
        return 0
    lax.fori_loop(0, _CH2, body, 0)

    def drain(j, _):
        pltpu.make_async_copy(rows_v, acc.at[idx_v.at[0]], sem).wait()
        return 0
    lax.fori_loop(0, 8, drain, 0)
    plsc.subcore_barrier()
    _acc_to_hbm(acc, rows_v, _CW, deg_hbm, c, base)


def _deg_call(src16, dst16):
    return pl.kernel(
        _deg_body,
        out_type=jax.ShapeDtypeStruct((_NC, _NP, _D), jnp.float32),
        mesh=_sc_mesh(),
        scratch_types=[
            pltpu.VMEM((_CH2, _CW), jnp.int32),
            pltpu.VMEM((_CW, _D), jnp.float32),
            pltpu.VMEM_SHARED((_NP, _D), jnp.float32),
            pltpu.SemaphoreType.DMA,
        ],
    )(src16, dst16)


_GW = 64
_GN = 160
_PAIRS = 80
_EPT = _GN * _GW


def _scat_body(h_hbm, src_hbm, dst_hbm, out_hbm, rows_big, ib, db, acc,
               sem_i, sem_g, sem_d):
    c = lax.axis_index("c")
    s = lax.axis_index("s")
    wid = c * _NS + s
    ebase = wid * _EPT
    _zero_rows(rows_big, 128)
    base = s * _RPT
    _zero_acc_slice(rows_big, 128, acc, base)

    pltpu.async_copy(dst_hbm.at[wid].at[0], db.at[0], sem_d)
    pltpu.async_copy(dst_hbm.at[wid].at[1], db.at[1], sem_d)
    plsc.subcore_barrier()

    def step(t, _):
        @pl.when(t < _GN)
        def _():
            slot = lax.rem(t, 6)
            pltpu.async_copy(src_hbm.at[pl.ds(ebase + t * _GW, _GW)],
                             ib.at[pl.ds(slot * _GW, _GW)], sem_i)

        @pl.when(jnp.logical_and(t >= 2, t < _GN + 2))
        def _():
            j = t - 2
            islot = lax.rem(j, 6)
            gslot = lax.rem(j, 4)
            pltpu.make_async_copy(src_hbm.at[pl.ds(0, _GW)],
                                  ib.at[pl.ds(0, _GW)], sem_i).wait()
            pltpu.async_copy(h_hbm.at[ib.at[pl.ds(islot * _GW, _GW)]],
                             rows_big.at[pl.ds(gslot * _GW, _GW)], sem_g)

        @pl.when(jnp.logical_and(t >= 5, lax.rem(t, 2) == 1))
        def _():
            m = (t - 5) // 2
            ms = lax.rem(m, 2)
            pltpu.make_async_copy(h_hbm.at[pl.ds(0, 128)],
                                  rows_big.at[pl.ds(0, 128)], sem_g).wait()
            pltpu.make_async_copy(dst_hbm.at[wid].at[0], db.at[0], sem_d).wait()
            pltpu.sync_copy(rows_big.at[pl.ds(ms * 128, 128)],
                            acc.at[db.at[ms]], add=True)

            @pl.when(m + 2 < _PAIRS)
            def _():
                pltpu.async_copy(dst_hbm.at[wid].at[m + 2], db.at[ms], sem_d)
        return 0
    lax.fori_loop(0, 2 * _PAIRS + 4, step, 0)
    plsc.subcore_barrier()
    _acc_to_hbm(acc, rows_big, 128, out_hbm, c, base)


def _scat_call(h, src1, dst3):
    return pl.kernel(
        _scat_body,
        out_type=jax.ShapeDtypeStruct((_NC, _NP, _D), jnp.float32),
        mesh=_sc_mesh(),
        scratch_types=[
            pltpu.VMEM((4 * _GW, _D), jnp.float32),
            pltpu.VMEM((6 * _GW,), jnp.int32),
            pltpu.VMEM((2, 128), jnp.int32),
            pltpu.VMEM_SHARED((_NP, _D), jnp.float32),
            pltpu.SemaphoreType.DMA,
            pltpu.SemaphoreType.DMA,
            pltpu.SemaphoreType.DMA,
        ],
    )(h, src1, dst3)


def _pair_body(h_hbm, g1_hbm, g2_hbm, o1_hbm, o2_hbm, g_v, rows_v):
    c = lax.axis_index("c")
    s = lax.axis_index("s")
    wid = c * _NS + s
    for g_hbm, o_hbm in ((g1_hbm, o1_hbm), (g2_hbm, o2_hbm)):
        pltpu.sync_copy(g_hbm.at[wid], g_v)
        pltpu.sync_copy(h_hbm.at[g_v], rows_v)
        pltpu.sync_copy(rows_v, o_hbm.at[pl.ds(wid * _CW, _CW)])


def _pair_call(h, g1, g2):
    return pl.kernel(
        _pair_body,
        out_type=(jax.ShapeDtypeStruct((_B, _D), jnp.float32),
                  jax.ShapeDtypeStruct((_B, _D), jnp.float32)),
        mesh=_sc_mesh(),
        scratch_types=[
            pltpu.VMEM((_CW,), jnp.int32),
            pltpu.VMEM((_CW, _D), jnp.float32),
        ],
    )(h, g1, g2)


_BLK = 2528


def _l1_body(x_ref, do_ref, w_ref, o_ref):
    ns = lax.rsqrt(jnp.maximum(do_ref[...], 1.0))
    o_ref[...] = jnp.dot(x_ref[...], w_ref[...],
                         preferred_element_type=jnp.float32) * ns


def _l1_call(x_p, deg_o, w):
    return pl.pallas_call(
        _l1_body,
        grid=(4,),
        in_specs=[
            pl.BlockSpec((_BLK, _D), lambda i: (i, 0)),
            pl.BlockSpec((_BLK, _D), lambda i: (i, 0)),
            pl.BlockSpec((_D, _D), lambda i: (0, 0)),
        ],
        out_specs=pl.BlockSpec((_BLK, _D), lambda i: (i, 0)),
        out_shape=jax.ShapeDtypeStruct((_NP, _D), jnp.float32),
    )(x_p, deg_o, w)


def _l23_body(s_ref, di_ref, do_ref, b_ref, w_ref, o_ref):
    nd = lax.rsqrt(jnp.maximum(di_ref[...], 1.0))
    ns = lax.rsqrt(jnp.maximum(do_ref[...], 1.0))
    h = jnp.maximum((s_ref[0] + s_ref[1]) * nd + b_ref[...], 0.0)
    o_ref[...] = jnp.dot(h, w_ref[...],
                         preferred_element_type=jnp.float32) * ns


def _l23_call(s, deg_i, deg_o, b, w):
    return pl.pallas_call(
        _l23_body,
        grid=(4,),
        in_specs=[
            pl.BlockSpec((_NC, _BLK, _D), lambda i: (0, i, 0)),
            pl.BlockSpec((_BLK, _D), lambda i: (i, 0)),
            pl.BlockSpec((_BLK, _D), lambda i: (i, 0)),
            pl.BlockSpec((1, _D), lambda i: (0, 0)),
            pl.BlockSpec((_D, _D), lambda i: (0, 0)),
        ],
        out_specs=pl.BlockSpec((_BLK, _D), lambda i: (i, 0)),
        out_shape=jax.ShapeDtypeStruct((_NP, _D), jnp.float32),
    )(s, deg_i, deg_o, b, w)


def _fin_body(s_ref, di_ref, b_ref, o_ref):
    nd = lax.rsqrt(jnp.maximum(di_ref[...], 1.0))
    o_ref[...] = (s_ref[0] + s_ref[1]) * nd + b_ref[...]


def _fin_call(s, deg_i, b):
    return pl.pallas_call(
        _fin_body,
        grid=(4,),
        in_specs=[
            pl.BlockSpec((_NC, _BLK, _D), lambda i: (0, i, 0)),
            pl.BlockSpec((_BLK, _D), lambda i: (i, 0)),
            pl.BlockSpec((1, _D), lambda i: (0, 0)),
        ],
        out_specs=pl.BlockSpec((_BLK, _D), lambda i: (i, 0)),
        out_shape=jax.ShapeDtypeStruct((_NP, _D), jnp.float32),
    )(s, deg_i, b)


def _mlp_body(g1_ref, g2_ref, a_ref, bm_ref, b1_ref, w2_ref, b2_ref, o_ref):
    z = jnp.dot(g1_ref[...], a_ref[...], preferred_element_type=jnp.float32)
    z = z + jnp.dot(g2_ref[...], bm_ref[...], preferred_element_type=jnp.float32)
    z = jnp.maximum(z + b1_ref[...], 0.0)
    lp = jnp.dot(z, w2_ref[...], preferred_element_type=jnp.float32) + b2_ref[...]
    l0 = lp[:, 0:1]
    l1 = lp[:, 1:2]
    m = jnp.maximum(l0, l1)
    e0 = jnp.exp(l0 - m)
    e1 = jnp.exp(l1 - m)
    tot = e0 + e1
    col = lax.broadcasted_iota(jnp.int32, (_B, _D), 1)
    o_ref[...] = jnp.where(col == 0, e0 / tot, jnp.where(col == 1, e1 / tot, 0.0))


def _mlp_call(r1, r2, a, bm, b1, w2p, b2p):
    return pl.pallas_call(
        _mlp_body,
        out_shape=jax.ShapeDtypeStruct((_B, _D), jnp.float32),
    )(r1, r2, a, bm, b1, w2p, b2p)


def kernel(x, edge_index, gene1, gene2, W1, b1, W2, b2, W3, b3,
           fc1_W, fc1_b, fc2_W, fc2_b):
    src = edge_index[0]
    dst = edge_index[1]
    pad_idx = _N + jnp.arange(_EP - _E, dtype=jnp.int32) % (_NP - _N)
    src_p = jnp.concatenate([src, pad_idx])
    dst_p = jnp.concatenate([dst, pad_idx])
    src1 = src_p
    dst3 = dst_p.reshape(_NW, _PAIRS, _CW)
    src16 = src_p.reshape(_NS, _CH2, _CW)
    dst16 = dst_p.reshape(_NS, _CH2, _CW)
    x_p = jnp.pad(x, ((0, _NP - _N), (0, 0)))
    g1 = gene1.reshape(_NW, _CW)
    g2 = gene2.reshape(_NW, _CW)

    deg = _deg_call(src16, dst16)
    deg_o = deg[0]
    deg_i = deg[1]

    h = _l1_call(x_p, deg_o, W1)
    s1 = _scat_call(h, src1, dst3)
    h = _l23_call(s1, deg_i, deg_o, b1.reshape(1, _D), W2)
    s2 = _scat_call(h, src1, dst3)
    h = _l23_call(s2, deg_i, deg_o, b2.reshape(1, _D), W3)
    s3 = _scat_call(h, src1, dst3)
    hf = _fin_call(s3, deg_i, b3.reshape(1, _D))

    r1, r2 = _pair_call(hf, g1, g2)

    a = fc1_W[:_D]
    bm = fc1_W[_D:]
    w2p = jnp.zeros((_D, _D), jnp.float32).at[:, :2].set(fc2_W)
    b2p = jnp.zeros((_D,), jnp.float32).at[:2].set(fc2_b)
    probs_p = _mlp_call(r1, r2, a, bm, fc1_b.reshape(1, _D),
                        w2p, b2p.reshape(1, _D))
    return probs_p[:, :2]
